# Initial kernel scaffold; baseline (speedup 1.0000x reference)
import jax
import jax.numpy as jnp
from jax.experimental import pallas as pl


def kernel(x, edge_index, edge_attr, fc1_1_w, fc1_2_w, fc1_3_w, fc1_4_w, conv_weight, conv_bias, fc11_w, fc11_b, fc12_w, fc12_b):
    raise NotImplementedError("write your pallas kernel here")


# speedup vs baseline: 4.9479x; 4.9479x over previous
import functools

import jax
import jax.numpy as jnp
from jax import lax
from jax.experimental import pallas as pl
from jax.experimental.pallas import tpu as pltpu
from jax.experimental.pallas import tpu_sc as plsc

_NC, _NS, _LANES = 2, 16, 16
_NW = _NC * _NS


def _edge_mlp_body(attr_ref, w123t_ref, w4t_ref, ea_ref):
    t = jnp.dot(attr_ref[...], w123t_ref[...], preferred_element_type=jnp.float32)
    h = jax.nn.relu(t[:, :32])
    g = jnp.tanh(t[:, 32:64]) * jnp.tanh(t[:, 64:96])
    tmp = jnp.concatenate([h, g], axis=1)
    ea_ref[...] = jax.nn.relu(
        jnp.dot(tmp, w4t_ref[...], preferred_element_type=jnp.float32))


def _project_body(x_ref, wz_ref, w11t_ref, b11_ref, w12t_ref, b12_ref,
                  z_ref, r_ref):
    x = x_ref[...]
    z_ref[...] = jnp.dot(x, wz_ref[...], preferred_element_type=jnp.float32)
    r_ref[...] = (
        jnp.tanh(jnp.dot(x, w11t_ref[...], preferred_element_type=jnp.float32)
                 + b11_ref[...])
        * jnp.tanh(jnp.dot(x, w12t_ref[...], preferred_element_type=jnp.float32)
                   + b12_ref[...]))


def _combine_body(p0_ref, p1_ref, bias_ref, r_ref, out_ref):
    left = jax.nn.relu(p0_ref[...] + p1_ref[...] + bias_ref[...])
    out_ref[...] = jnp.concatenate([left, r_ref[...]], axis=1)


def kernel(x, edge_index, edge_attr, fc1_1_w, fc1_2_w, fc1_3_w, fc1_4_w,
           conv_weight, conv_bias, fc11_w, fc11_b, fc12_w, fc12_b):
    n, ninp = x.shape
    e = edge_attr.shape[0]
    k_sup, _, nout1 = conv_weight.shape
    nout2 = fc11_w.shape[0]
    d = k_sup * nout1
    nf = nout1 // _LANES

    src = edge_index[0].astype(jnp.int32)
    dst = edge_index[1].astype(jnp.int32)
    w123t = jnp.concatenate([fc1_1_w, fc1_2_w, fc1_3_w], axis=0).T
    w4t = fc1_4_w.T
    wz = conv_weight.transpose(1, 0, 2).reshape(ninp, d)

    be = 4000
    ea = pl.pallas_call(
        _edge_mlp_body,
        grid=(e // be,),
        in_specs=[
            pl.BlockSpec((be, edge_attr.shape[1]), lambda i: (i, 0)),
            pl.BlockSpec(w123t.shape, lambda i: (0, 0)),
            pl.BlockSpec(w4t.shape, lambda i: (0, 0)),
        ],
        out_specs=pl.BlockSpec((be, k_sup), lambda i: (i, 0)),
        out_shape=jax.ShapeDtypeStruct((e, k_sup), jnp.float32),
    )(edge_attr, w123t, w4t)

    bn = 2000
    z, r = pl.pallas_call(
        _project_body,
        grid=(n // bn,),
        in_specs=[
            pl.BlockSpec((bn, ninp), lambda i: (i, 0)),
            pl.BlockSpec((ninp, d), lambda i: (0, 0)),
            pl.BlockSpec((ninp, nout2), lambda i: (0, 0)),
            pl.BlockSpec((1, nout2), lambda i: (0, 0)),
            pl.BlockSpec((ninp, nout2), lambda i: (0, 0)),
            pl.BlockSpec((1, nout2), lambda i: (0, 0)),
        ],
        out_specs=[
            pl.BlockSpec((bn, d), lambda i: (i, 0)),
            pl.BlockSpec((bn, nout2), lambda i: (i, 0)),
        ],
        out_shape=[
            jax.ShapeDtypeStruct((n, d), jnp.float32),
            jax.ShapeDtypeStruct((n, nout2), jnp.float32),
        ],
    )(x, wz, fc11_w.T, fc11_b.reshape(1, -1), fc12_w.T, fc12_b.reshape(1, -1))

    chunk = 8
    e_per_w = e // _NW
    nch = e_per_w // chunk
    n_pad = ((n + 8 * _NS - 1) // (8 * _NS)) * (8 * _NS)
    rows_per_s = n_pad // _NS
    zb = 64
    nzb = rows_per_s // zb

    mesh = plsc.VectorSubcoreMesh(core_axis_name="c", subcore_axis_name="s")

    @functools.partial(
        pl.kernel,
        out_type=jax.ShapeDtypeStruct((_NC, n_pad, nout1), jnp.float32),
        mesh=mesh,
        scratch_types=[
            pltpu.VMEM((chunk,), jnp.int32),
            pltpu.VMEM((chunk,), jnp.int32),
            pltpu.VMEM((chunk, k_sup), jnp.float32),
            pltpu.VMEM((chunk, d), jnp.float32),
            pltpu.VMEM((chunk, nout1), jnp.float32),
            pltpu.VMEM((zb, nout1), jnp.float32),
            pltpu.VMEM_SHARED((n_pad, nout1), jnp.float32),
            pltpu.SemaphoreType.DMA,
        ],
    )
    def _sc_spect(src_hbm, dst_hbm, ea_hbm, z_hbm, out_hbm,
                  src_v, dst_v, ea_v, z_v, y_v, zero_v, acc_sh, sem):
        cid = lax.axis_index("c")
        sid = lax.axis_index("s")
        wid = sid * _NC + cid
        zvec = jnp.zeros((_LANES,), jnp.float32)

        def _zero_row(rr, carry):
            for f in range(nf):
                zero_v[rr, pl.ds(f * _LANES, _LANES)] = zvec
            return carry

        lax.fori_loop(0, zb, _zero_row, 0)
        for j in range(nzb):
            pltpu.sync_copy(
                zero_v, acc_sh.at[pl.ds(sid * rows_per_s + j * zb, zb)])
        plsc.subcore_barrier()

        def _chunk_body(kc, carry):
            base = pl.multiple_of(wid * e_per_w + kc * chunk, 8)
            pltpu.sync_copy(src_hbm.at[pl.ds(base, chunk)], src_v)
            pltpu.sync_copy(dst_hbm.at[pl.ds(base, chunk)], dst_v)
            pltpu.sync_copy(ea_hbm.at[pl.ds(base, chunk)], ea_v)
            pltpu.async_copy(z_hbm.at[src_v], z_v, sem).wait()

            def _edge(ee, ecarry):
                ea_row = ea_v[ee, :]
                accs = [zvec] * nf
                for i in range(k_sup):
                    a = ea_row.at[jnp.full((_LANES,), i, jnp.int32)].get(
                        mode="promise_in_bounds")
                    for f in range(nf):
                        accs[f] = accs[f] + a * z_v[
                            ee, pl.ds(i * nout1 + f * _LANES, _LANES)]
                for f in range(nf):
                    y_v[ee, pl.ds(f * _LANES, _LANES)] = accs[f]
                return ecarry

            lax.fori_loop(0, chunk, _edge, 0)
            pltpu.sync_copy(y_v, acc_sh.at[dst_v], add=True)
            return carry

        lax.fori_loop(0, nch, _chunk_body, 0)

        plsc.subcore_barrier()
        pltpu.sync_copy(acc_sh.at[pl.ds(sid * rows_per_s, rows_per_s)],
                        out_hbm.at[cid, pl.ds(sid * rows_per_s, rows_per_s)])

    partials = _sc_spect(src, dst, ea, z)[:, :n, :]

    out = pl.pallas_call(
        _combine_body,
        grid=(n // bn,),
        in_specs=[
            pl.BlockSpec((bn, nout1), lambda i: (i, 0)),
            pl.BlockSpec((bn, nout1), lambda i: (i, 0)),
            pl.BlockSpec((1, nout1), lambda i: (0, 0)),
            pl.BlockSpec((bn, nout2), lambda i: (i, 0)),
        ],
        out_specs=pl.BlockSpec((bn, nout1 + nout2), lambda i: (i, 0)),
        out_shape=jax.ShapeDtypeStruct((n, nout1 + nout2), jnp.float32),
    )(partials[0], partials[1], conv_bias.reshape(1, -1), r)
    return out


# speedup vs baseline: 7.5015x; 1.5161x over previous
import functools

import jax
import jax.numpy as jnp
from jax import lax
from jax.experimental import pallas as pl
from jax.experimental.pallas import tpu as pltpu
from jax.experimental.pallas import tpu_sc as plsc

_NC, _NS, _LANES = 2, 16, 16
_NW = _NC * _NS


def _edge_mlp_body(attr_ref, w123t_ref, w4t_ref, ea_ref):
    t = jnp.dot(attr_ref[...], w123t_ref[...], preferred_element_type=jnp.float32)
    h = jax.nn.relu(t[:, :32])
    g = jnp.tanh(t[:, 32:64]) * jnp.tanh(t[:, 64:96])
    tmp = jnp.concatenate([h, g], axis=1)
    ea_ref[...] = jax.nn.relu(
        jnp.dot(tmp, w4t_ref[...], preferred_element_type=jnp.float32))


def _project_body(x_ref, wz_ref, w11t_ref, b11_ref, w12t_ref, b12_ref,
                  z_ref, r_ref):
    x = x_ref[...]
    z_ref[...] = jnp.dot(x, wz_ref[...], preferred_element_type=jnp.float32)
    r_ref[...] = (
        jnp.tanh(jnp.dot(x, w11t_ref[...], preferred_element_type=jnp.float32)
                 + b11_ref[...])
        * jnp.tanh(jnp.dot(x, w12t_ref[...], preferred_element_type=jnp.float32)
                   + b12_ref[...]))


def _combine_body(p0_ref, p1_ref, bias_ref, r_ref, out_ref):
    left = jax.nn.relu(p0_ref[...] + p1_ref[...] + bias_ref[...])
    out_ref[...] = jnp.concatenate([left, r_ref[...]], axis=1)


def kernel(x, edge_index, edge_attr, fc1_1_w, fc1_2_w, fc1_3_w, fc1_4_w,
           conv_weight, conv_bias, fc11_w, fc11_b, fc12_w, fc12_b):
    n, ninp = x.shape
    e = edge_attr.shape[0]
    k_sup, _, nout1 = conv_weight.shape
    nout2 = fc11_w.shape[0]
    d = k_sup * nout1
    nf = nout1 // _LANES

    src = edge_index[0].astype(jnp.int32)
    dst = edge_index[1].astype(jnp.int32)
    w123t = jnp.concatenate([fc1_1_w, fc1_2_w, fc1_3_w], axis=0).T
    w4t = fc1_4_w.T
    wz = conv_weight.transpose(1, 0, 2).reshape(ninp, d)

    be = 4000
    ea = pl.pallas_call(
        _edge_mlp_body,
        grid=(e // be,),
        in_specs=[
            pl.BlockSpec((be, edge_attr.shape[1]), lambda i: (i, 0)),
            pl.BlockSpec(w123t.shape, lambda i: (0, 0)),
            pl.BlockSpec(w4t.shape, lambda i: (0, 0)),
        ],
        out_specs=pl.BlockSpec((be, k_sup), lambda i: (i, 0)),
        out_shape=jax.ShapeDtypeStruct((e, k_sup), jnp.float32),
    )(edge_attr, w123t, w4t)

    bn = 2000
    z, r = pl.pallas_call(
        _project_body,
        grid=(n // bn,),
        in_specs=[
            pl.BlockSpec((bn, ninp), lambda i: (i, 0)),
            pl.BlockSpec((ninp, d), lambda i: (0, 0)),
            pl.BlockSpec((ninp, nout2), lambda i: (0, 0)),
            pl.BlockSpec((1, nout2), lambda i: (0, 0)),
            pl.BlockSpec((ninp, nout2), lambda i: (0, 0)),
            pl.BlockSpec((1, nout2), lambda i: (0, 0)),
        ],
        out_specs=[
            pl.BlockSpec((bn, d), lambda i: (i, 0)),
            pl.BlockSpec((bn, nout2), lambda i: (i, 0)),
        ],
        out_shape=[
            jax.ShapeDtypeStruct((n, d), jnp.float32),
            jax.ShapeDtypeStruct((n, nout2), jnp.float32),
        ],
    )(x, wz, fc11_w.T, fc11_b.reshape(1, -1), fc12_w.T, fc12_b.reshape(1, -1))

    chunk = 8
    sup = 40
    cps = sup // chunk
    e_per_w = e // _NW
    nsup = e_per_w // sup
    n_pad = ((n + 8 * _NS - 1) // (8 * _NS)) * (8 * _NS)
    rows_per_s = n_pad // _NS
    nzb = rows_per_s // sup

    mesh = plsc.VectorSubcoreMesh(core_axis_name="c", subcore_axis_name="s")

    @functools.partial(
        pl.kernel,
        out_type=jax.ShapeDtypeStruct((_NC, n_pad, nout1), jnp.float32),
        mesh=mesh,
        scratch_types=[
            pltpu.VMEM((sup,), jnp.int32),
            pltpu.VMEM((sup,), jnp.int32),
            pltpu.VMEM((sup, k_sup), jnp.float32),
            pltpu.VMEM((chunk, d), jnp.float32),
            pltpu.VMEM((chunk, d), jnp.float32),
            pltpu.VMEM((chunk, nout1), jnp.float32),
            pltpu.VMEM((chunk,), jnp.int32),
            pltpu.VMEM((chunk,), jnp.int32),
            pltpu.VMEM((chunk,), jnp.int32),
            pltpu.VMEM_SHARED((n_pad, nout1), jnp.float32),
            pltpu.SemaphoreType.DMA,
            pltpu.SemaphoreType.DMA,
        ],
    )
    def _sc_spect(src_hbm, dst_hbm, ea_hbm, z_hbm, out_hbm,
                  src_v, dst_v, ea_v, z_a, z_b, y_v, idx_a, idx_b, dbuf,
                  acc_sh, sem_a, sem_b):
        cid = lax.axis_index("c")
        sid = lax.axis_index("s")
        wid = sid * _NC + cid
        zvec = jnp.zeros((_LANES,), jnp.float32)
        zbufs = (z_a, z_b)
        sems = (sem_a, sem_b)
        ibufs = (idx_a, idx_b)

        def _zero_row(rr, carry):
            for f in range(nf):
                y_v[rr, pl.ds(f * _LANES, _LANES)] = zvec
            return carry

        lax.fori_loop(0, chunk, _zero_row, 0)

        def _zero_cp(j, carry):
            pltpu.sync_copy(
                y_v, acc_sh.at[pl.ds(sid * rows_per_s + j * chunk, chunk)])
            return carry

        lax.fori_loop(0, rows_per_s // chunk, _zero_cp, 0)
        plsc.subcore_barrier()

        def _sup_body(sc, carry):
            base = pl.multiple_of(wid * e_per_w + sc * sup, 8)
            pltpu.sync_copy(ea_hbm.at[pl.ds(base, sup)], ea_v)

            descs = [None] * cps
            pltpu.sync_copy(src_hbm.at[pl.ds(base, chunk)], ibufs[0])
            descs[0] = pltpu.async_copy(
                z_hbm.at[ibufs[0]], zbufs[0], sems[0])
            for kc in range(cps):
                if kc + 1 < cps:
                    b1 = (kc + 1) % 2
                    pltpu.sync_copy(
                        src_hbm.at[pl.ds(base + (kc + 1) * chunk, chunk)],
                        ibufs[b1])
                    descs[kc + 1] = pltpu.async_copy(
                        z_hbm.at[ibufs[b1]], zbufs[b1], sems[b1])
                descs[kc].wait()
                zv = zbufs[kc % 2]

                def _edge(ee, ecarry, _kc=kc, _zv=zv):
                    ea_row = ea_v[_kc * chunk + ee, :]
                    accs = [zvec] * nf
                    for i in range(k_sup):
                        a = ea_row.at[jnp.full((_LANES,), i, jnp.int32)].get(
                            mode="promise_in_bounds")
                        for f in range(nf):
                            accs[f] = accs[f] + a * _zv[
                                ee, pl.ds(i * nout1 + f * _LANES, _LANES)]
                    for f in range(nf):
                        y_v[ee, pl.ds(f * _LANES, _LANES)] = accs[f]
                    return ecarry

                lax.fori_loop(0, chunk, _edge, 0)
                pltpu.sync_copy(
                    dst_hbm.at[pl.ds(base + kc * chunk, chunk)], dbuf)
                pltpu.sync_copy(y_v, acc_sh.at[dbuf], add=True)
            return carry

        lax.fori_loop(0, nsup, _sup_body, 0)

        plsc.subcore_barrier()
        pltpu.sync_copy(acc_sh.at[pl.ds(sid * rows_per_s, rows_per_s)],
                        out_hbm.at[cid, pl.ds(sid * rows_per_s, rows_per_s)])

    partials = _sc_spect(src, dst, ea, z)[:, :n, :]

    out = pl.pallas_call(
        _combine_body,
        grid=(n // bn,),
        in_specs=[
            pl.BlockSpec((bn, nout1), lambda i: (i, 0)),
            pl.BlockSpec((bn, nout1), lambda i: (i, 0)),
            pl.BlockSpec((1, nout1), lambda i: (0, 0)),
            pl.BlockSpec((bn, nout2), lambda i: (i, 0)),
        ],
        out_specs=pl.BlockSpec((bn, nout1 + nout2), lambda i: (i, 0)),
        out_shape=jax.ShapeDtypeStruct((n, nout1 + nout2), jnp.float32),
    )(partials[0], partials[1], conv_bias.reshape(1, -1), r)
    return out
